# baseline (speedup 1.0000x reference)
import jax
import jax.numpy as jnp
from jax.experimental import pallas as pl
from jax.experimental.pallas import tpu as pltpu

_BINS = 100
_ROWS = 16


def _hist_loss_kernel(f0_ref, f1_ref, gt_ref, out_ref, h_ref):
    i = pl.program_id(0)
    nsteps = pl.num_programs(0)

    @pl.when(i == 0)
    def _init():
        h_ref[...] = jnp.zeros_like(h_ref)

    c = f0_ref.shape[0]
    r, w = gt_ref.shape
    dist2 = jnp.zeros((r, w), jnp.float32)
    for k0 in range(0, c, 8):
        t = f0_ref[k0:k0 + 8] + 1e-6 - f1_ref[k0:k0 + 8]
        dist2 = dist2 + jnp.sum(t * t, axis=0)
    dist = jnp.sqrt(dist2)
    gt = gt_ref[...]
    raw = jnp.minimum(jnp.floor(dist * float(_BINS)), float(_BINS)).astype(jnp.int32)
    idx = jnp.where(dist <= 1.0, jnp.minimum(raw, _BINS - 1), _BINS)
    val = jnp.where(gt == 0, 1, 1 << 16)
    bins = jax.lax.broadcasted_iota(jnp.int32, (_BINS + 1, 1, 1), 0)
    onehot = idx[None, :, :] == bins
    h_ref[...] += jnp.sum(jnp.where(onehot, val[None, :, :], 0), axis=1)

    @pl.when(i == nsteps - 1)
    def _finalize():
        npix = nsteps * _ROWS * gt_ref.shape[-1]
        hcells = h_ref[...]
        pcells = (hcells & 0xFFFF).astype(jnp.float32)
        ncells = (hcells >> 16).astype(jnp.float32)
        pos_size = jnp.sum(pcells)
        neg_size = float(npix) - pos_size
        hps = jnp.sum(pcells[0:_BINS], axis=1, keepdims=True)
        hns = jnp.sum(ncells[0:_BINS], axis=1, keepdims=True)
        hp = hps / pos_size
        hn = hns / neg_size
        pointwise = jnp.where(hn > 0, hn * (jnp.log(hn) - hp), 0.0)
        out_ref[...] = (jnp.sum(pointwise) / float(_BINS) + 1.0).reshape(1, 1)


@jax.jit
def kernel(feat_t0, feat_t1, ground_truth):
    n, c, h, w = feat_t0.shape
    f0 = feat_t0.reshape(c, h, w)
    f1 = feat_t1.reshape(c, h, w)
    grid = h // _ROWS
    out = pl.pallas_call(
        _hist_loss_kernel,
        grid=(grid,),
        in_specs=[
            pl.BlockSpec((c, _ROWS, w), lambda i: (0, i, 0)),
            pl.BlockSpec((c, _ROWS, w), lambda i: (0, i, 0)),
            pl.BlockSpec((_ROWS, w), lambda i: (i, 0)),
        ],
        out_specs=pl.BlockSpec((1, 1), lambda i: (0, 0)),
        out_shape=jax.ShapeDtypeStruct((1, 1), jnp.float32),
        scratch_shapes=[
            pltpu.VMEM((_BINS + 1, w), jnp.int32),
        ],
        compiler_params=pltpu.CompilerParams(
            dimension_semantics=("arbitrary",),
        ),
    )(f0, f1, ground_truth)
    return out[0, 0]


# speedup vs baseline: 1.1277x; 1.1277x over previous
import jax
import jax.numpy as jnp
from jax.experimental import pallas as pl
from jax.experimental.pallas import tpu as pltpu

_BINS = 100
_ROWS = 32


def _hist_loss_kernel(f0_ref, f1_ref, gt_ref, out_ref, h_ref):
    i = pl.program_id(0)
    nsteps = pl.num_programs(0)

    @pl.when(i == 0)
    def _init():
        h_ref[...] = jnp.zeros_like(h_ref)

    c = f0_ref.shape[0]
    r, w = gt_ref.shape
    dist2 = jnp.zeros((r, w), jnp.float32)
    for k0 in range(0, c, 8):
        t = f0_ref[k0:k0 + 8] + 1e-6 - f1_ref[k0:k0 + 8]
        dist2 = dist2 + jnp.sum(t * t, axis=0)
    dist = jnp.sqrt(dist2)
    gt = gt_ref[...]
    raw = jnp.minimum(jnp.floor(dist * float(_BINS)), float(_BINS)).astype(jnp.int32)
    idx = jnp.where(dist <= 1.0, jnp.minimum(raw, _BINS - 1), _BINS)
    val = jnp.where(gt == 0, 1, 1 << 16)
    bins = jax.lax.broadcasted_iota(jnp.int32, (_BINS + 1, 1, 1), 0)
    onehot = idx[None, :, :] == bins
    h_ref[...] += jnp.sum(jnp.where(onehot, val[None, :, :], 0), axis=1)

    @pl.when(i == nsteps - 1)
    def _finalize():
        npix = nsteps * _ROWS * gt_ref.shape[-1]
        hcells = h_ref[...]
        pcells = (hcells & 0xFFFF).astype(jnp.float32)
        ncells = (hcells >> 16).astype(jnp.float32)
        pos_size = jnp.sum(pcells)
        neg_size = float(npix) - pos_size
        hps = jnp.sum(pcells[0:_BINS], axis=1, keepdims=True)
        hns = jnp.sum(ncells[0:_BINS], axis=1, keepdims=True)
        hp = hps / pos_size
        hn = hns / neg_size
        pointwise = jnp.where(hn > 0, hn * (jnp.log(hn) - hp), 0.0)
        out_ref[...] = (jnp.sum(pointwise) / float(_BINS) + 1.0).reshape(1, 1)


@jax.jit
def kernel(feat_t0, feat_t1, ground_truth):
    n, c, h, w = feat_t0.shape
    f0 = feat_t0.reshape(c, h, w)
    f1 = feat_t1.reshape(c, h, w)
    grid = h // _ROWS
    out = pl.pallas_call(
        _hist_loss_kernel,
        grid=(grid,),
        in_specs=[
            pl.BlockSpec((c, _ROWS, w), lambda i: (0, i, 0)),
            pl.BlockSpec((c, _ROWS, w), lambda i: (0, i, 0)),
            pl.BlockSpec((_ROWS, w), lambda i: (i, 0)),
        ],
        out_specs=pl.BlockSpec((1, 1), lambda i: (0, 0)),
        out_shape=jax.ShapeDtypeStruct((1, 1), jnp.float32),
        scratch_shapes=[
            pltpu.VMEM((_BINS + 1, w), jnp.int32),
        ],
        compiler_params=pltpu.CompilerParams(
            dimension_semantics=("arbitrary",),
        ),
    )(f0, f1, ground_truth)
    return out[0, 0]
